# baseline (speedup 1.0000x reference)
import functools

import jax
import jax.numpy as jnp
from jax import lax
from jax.experimental import pallas as pl
from jax.experimental.pallas import tpu as pltpu
from jax.experimental.pallas import tpu_sc as plsc

DM = 1024
DF = 4096
NE = 8
NK = 2
LBW = 0.01
T = 4096
NP = T * NK

BLK = 512
NB = NP // BLK + NE
NPAD = NB * BLK

EP = 128

NC, NS = 2, 16
NW = NC * NS

_INTERPRET = False


TG = 512
NT = T // TG


def _gate_body(x_ref, gw_ref, gb_ref, gp_ref, tki_ref, tkw_ref,
               f_ref, p_ref, loss_ref):
    t = pl.program_id(0)
    logits = jnp.dot(x_ref[...], gw_ref[...],
                     preferred_element_type=jnp.float32) + gb_ref[...]
    lane = lax.broadcasted_iota(jnp.int32, (TG, EP), 1)
    big_neg = jnp.float32(-1e30)
    masked = jnp.where(lane < NE, logits, big_neg)
    m1 = jnp.max(masked, axis=1, keepdims=True)
    i1 = jnp.min(jnp.where(masked == m1, lane, NE), axis=1, keepdims=True)
    masked2 = jnp.where(lane == i1, big_neg, masked)
    m2 = jnp.max(masked2, axis=1, keepdims=True)
    i2 = jnp.min(jnp.where(masked2 == m2, lane, NE), axis=1, keepdims=True)

    z = jnp.exp(masked - m1)
    s = jnp.sum(z, axis=1, keepdims=True)
    probs = z / s
    gp_ref[...] = probs

    p1 = 1.0 / s
    p2 = jnp.exp(m2 - m1) / s
    denom = p1 + p2
    tki_ref[...] = jnp.concatenate([i1, i2], axis=1)
    tkw_ref[...] = jnp.concatenate([p1 / denom, p2 / denom], axis=1)

    @pl.when(t == 0)
    def _():
        f_ref[...] = jnp.zeros_like(f_ref)
        p_ref[...] = jnp.zeros_like(p_ref)

    f_ref[...] += jnp.sum((lane == i1).astype(jnp.float32), axis=0,
                          keepdims=True)
    p_ref[...] += jnp.sum(probs, axis=0, keepdims=True)

    @pl.when(t == NT - 1)
    def _():
        loss_ref[...] = jnp.reshape(
            (LBW * NE / (T * T)) * jnp.sum(f_ref[...] * p_ref[...]), (1, 1))


def _gate_call(x_flat, gw_pad, gb_pad):
    return pl.pallas_call(
        _gate_body,
        grid=(NT,),
        in_specs=[
            pl.BlockSpec((TG, DM), lambda t: (t, 0)),
            pl.BlockSpec((DM, EP), lambda t: (0, 0)),
            pl.BlockSpec((1, EP), lambda t: (0, 0)),
        ],
        out_specs=[
            pl.BlockSpec((TG, EP), lambda t: (t, 0)),
            pl.BlockSpec((TG, NK), lambda t: (t, 0)),
            pl.BlockSpec((TG, NK), lambda t: (t, 0)),
            pl.BlockSpec((1, EP), lambda t: (0, 0)),
            pl.BlockSpec((1, EP), lambda t: (0, 0)),
            pl.BlockSpec((1, 1), lambda t: (0, 0)),
        ],
        out_shape=[
            jax.ShapeDtypeStruct((T, EP), jnp.float32),
            jax.ShapeDtypeStruct((T, NK), jnp.int32),
            jax.ShapeDtypeStruct((T, NK), jnp.float32),
            jax.ShapeDtypeStruct((1, EP), jnp.float32),
            jax.ShapeDtypeStruct((1, EP), jnp.float32),
            jax.ShapeDtypeStruct((1, 1), jnp.float32),
        ],
        interpret=_INTERPRET,
    )(x_flat, gw_pad, gb_pad)


def _routing_meta(tki, tkw):
    e_p = tki.reshape(-1)
    onehot = (e_p[:, None] == jnp.arange(NE)[None, :]).astype(jnp.int32)
    rank = jnp.sum((jnp.cumsum(onehot, axis=0) - onehot) * onehot, axis=1)
    counts = jnp.sum(onehot, axis=0)
    nblk_e = (counts + BLK - 1) // BLK
    cum = jnp.cumsum(nblk_e).astype(jnp.int32)
    blk_start = jnp.concatenate([jnp.zeros((1,), jnp.int32), cum[:-1]])
    pos = blk_start[e_p] * BLK + rank
    total_blocks = cum[-1]

    bid = jnp.arange(NB, dtype=jnp.int32)
    be_raw = jnp.sum((bid[:, None] >= blk_start[None, :]).astype(jnp.int32),
                     axis=1) - 1
    last_e = jnp.sum((jnp.maximum(total_blocks - 1, 0) >= blk_start)
                     .astype(jnp.int32)) - 1
    valid = bid < total_blocks
    block_expert = jnp.where(valid, be_raw, last_e)
    block_valid = valid.astype(jnp.int32)

    pos2 = pos.reshape(T, NK).astype(jnp.int32)
    inv1 = pos2[:, 0]
    inv2 = pos2[:, 1]
    return block_expert, block_valid, inv1, inv2


def _expert_body(be_ref, bv_ref, x_ref, w1_ref, b1_ref, w2_ref, b2_ref,
                 y_ref):
    b = pl.program_id(0)

    @pl.when(bv_ref[b] == 1)
    def _():
        xb = x_ref[...]
        h = jnp.dot(xb, w1_ref[0],
                    preferred_element_type=jnp.float32) + b1_ref[0]
        h = 0.5 * h * (1.0 + lax.erf(h * jnp.float32(0.7071067811865476)))
        hb = h.astype(jnp.bfloat16)
        y_ref[...] = jnp.dot(hb, w2_ref[0],
                             preferred_element_type=jnp.float32) + b2_ref[0]


def _expert_call(x_sorted, w1b, b1, w2b, b2, block_expert, block_valid):
    grid_spec = pltpu.PrefetchScalarGridSpec(
        num_scalar_prefetch=2,
        grid=(NB,),
        in_specs=[
            pl.BlockSpec((BLK, DM), lambda b, be, bv: (b, 0)),
            pl.BlockSpec((1, DM, DF), lambda b, be, bv: (be[b], 0, 0)),
            pl.BlockSpec((1, 1, DF), lambda b, be, bv: (be[b], 0, 0)),
            pl.BlockSpec((1, DF, DM), lambda b, be, bv: (be[b], 0, 0)),
            pl.BlockSpec((1, 1, DM), lambda b, be, bv: (be[b], 0, 0)),
        ],
        out_specs=pl.BlockSpec((BLK, DM), lambda b, be, bv: (b, 0)),
    )
    return pl.pallas_call(
        _expert_body,
        grid_spec=grid_spec,
        out_shape=jax.ShapeDtypeStruct((NPAD, DM), jnp.float32),
        compiler_params=pltpu.CompilerParams(
            dimension_semantics=("arbitrary",),
        ),
        interpret=_INTERPRET,
    )(block_expert, block_valid, x_sorted,
      w1b, b1.reshape(NE, 1, DF), w2b, b2.reshape(NE, 1, DM))


TW = T // NW
CT = 32
TCH = TW // CT

@functools.lru_cache(maxsize=None)
def _sc_mesh():
    return plsc.VectorSubcoreMesh(
        core_axis_name="c", subcore_axis_name="s",
        num_cores=NC, num_subcores=NS)


def _sc_dispatch_body(x_hbm, ia_hbm, ib_hbm, out_hbm,
                      ia_v, ib_v, rows_v, sem_a, sem_b):
    wid = lax.axis_index("s") * NC + lax.axis_index("c")

    def chunk(i, carry):
        pltpu.sync_copy(ia_hbm.at[wid, i], ia_v)
        pltpu.sync_copy(ib_hbm.at[wid, i], ib_v)
        pltpu.sync_copy(x_hbm.at[pl.ds(wid * TW + i * CT, CT)], rows_v)
        cp_a = pltpu.async_copy(rows_v, out_hbm.at[ia_v], sem_a)
        cp_b = pltpu.async_copy(rows_v, out_hbm.at[ib_v], sem_b)
        cp_a.wait()
        cp_b.wait()
        return carry

    lax.fori_loop(0, TCH, chunk, 0)


DM2 = DM // 2


def _sc_dispatch_call(x32_flat, inv1, inv2):
    ia = inv1.reshape(NW, TCH, CT)
    ib = inv2.reshape(NW, TCH, CT)
    return pl.kernel(
        _sc_dispatch_body,
        out_type=jax.ShapeDtypeStruct((NPAD, DM2), jnp.int32),
        mesh=_sc_mesh(),
        scratch_types=[
            pltpu.VMEM((CT,), jnp.int32),
            pltpu.VMEM((CT,), jnp.int32),
            pltpu.VMEM((CT, DM2), jnp.int32),
            pltpu.SemaphoreType.DMA,
            pltpu.SemaphoreType.DMA,
        ],
    )(x32_flat, ia, ib)


RC = T // NW
CC = 32


def _sc_combine_body(y_hbm, inv1_hbm, inv2_hbm, wa_hbm, wb_hbm, out_hbm,
                     i1_v, i2_v, wa_v, wb_v, r1_v, r2_v, sem1, sem2):
    wid = lax.axis_index("s") * NC + lax.axis_index("c")
    base = wid * RC

    def chunk(i, carry):
        off = base + i * CC
        pltpu.sync_copy(inv1_hbm.at[pl.ds(off, CC)], i1_v)
        pltpu.sync_copy(inv2_hbm.at[pl.ds(off, CC)], i2_v)
        pltpu.sync_copy(wa_hbm.at[pl.ds(off, CC)], wa_v)
        pltpu.sync_copy(wb_hbm.at[pl.ds(off, CC)], wb_v)
        cp1 = pltpu.async_copy(y_hbm.at[i1_v], r1_v, sem1)
        cp2 = pltpu.async_copy(y_hbm.at[i2_v], r2_v, sem2)
        cp1.wait()
        cp2.wait()

        def row(r, c2):
            wa = wa_v[r, :]
            wb = wb_v[r, :]
            for j in range(DM // 16):
                sl = pl.ds(j * 16, 16)
                r1_v[r, sl] = r1_v[r, sl] * wa + r2_v[r, sl] * wb
            return c2

        lax.fori_loop(0, CC, row, 0)
        pltpu.sync_copy(r1_v, out_hbm.at[pl.ds(off, CC)])
        return carry

    lax.fori_loop(0, RC // CC, chunk, 0)


def _sc_combine_call(y_sorted, inv1, inv2, wa, wb):
    return pl.kernel(
        _sc_combine_body,
        out_type=jax.ShapeDtypeStruct((T, DM), jnp.float32),
        mesh=_sc_mesh(),
        scratch_types=[
            pltpu.VMEM((CC,), jnp.int32),
            pltpu.VMEM((CC,), jnp.int32),
            pltpu.VMEM((CC, 16), jnp.float32),
            pltpu.VMEM((CC, 16), jnp.float32),
            pltpu.VMEM((CC, DM), jnp.float32),
            pltpu.VMEM((CC, DM), jnp.float32),
            pltpu.SemaphoreType.DMA,
            pltpu.SemaphoreType.DMA,
        ],
    )(y_sorted, inv1, inv2, wa, wb)


def kernel(x, gate_W, gate_b, W1, b1, W2, b2):
    B, S, D = x.shape
    x_flat = x.reshape(-1, D)

    gw_pad = jnp.zeros((DM, EP), jnp.float32).at[:, :NE].set(gate_W)
    gb_pad = jnp.zeros((1, EP), jnp.float32).at[0, :NE].set(gate_b)

    gp_pad, tki, tkw, _, _, loss = _gate_call(x_flat, gw_pad, gb_pad)

    block_expert, block_valid, inv1, inv2 = _routing_meta(tki, tkw)

    xb = x_flat.astype(jnp.bfloat16)
    x32 = lax.bitcast_convert_type(xb.reshape(T, DM2, 2), jnp.int32)
    xs32 = _sc_dispatch_call(x32, inv1, inv2)
    x_sorted = lax.bitcast_convert_type(
        xs32, jnp.bfloat16).reshape(NPAD, DM)
    y_sorted = _expert_call(x_sorted, W1.astype(jnp.bfloat16), b1,
                            W2.astype(jnp.bfloat16), b2,
                            block_expert, block_valid)
    out_flat = _sc_combine_call(
        y_sorted, inv1, inv2,
        jnp.broadcast_to(tkw[:, 0:1], (T, 16)),
        jnp.broadcast_to(tkw[:, 1:2], (T, 16)))

    return (out_flat.reshape(B, S, D), loss.reshape(()),
            gp_pad[:, :NE].reshape(B, S, NE), tki.reshape(B, S, NK))


# speedup vs baseline: 1.7817x; 1.7817x over previous
import functools

import jax
import jax.numpy as jnp
from jax import lax
from jax.experimental import pallas as pl
from jax.experimental.pallas import tpu as pltpu
from jax.experimental.pallas import tpu_sc as plsc

DM = 1024
DF = 4096
NE = 8
NK = 2
LBW = 0.01
T = 4096
NP = T * NK

BLK = 512
NB = NP // BLK + NE
NPAD = NB * BLK

EP = 128

NC, NS = 2, 16
NW = NC * NS

_INTERPRET = False


TG = 512
NT = T // TG


def _gate_body(x_ref, gw_ref, gb_ref, gp_ref, tki_ref, tkw_ref,
               f_ref, p_ref, loss_ref):
    t = pl.program_id(0)
    logits = jnp.dot(x_ref[...], gw_ref[...],
                     preferred_element_type=jnp.float32) + gb_ref[...]
    lane = lax.broadcasted_iota(jnp.int32, (TG, EP), 1)
    big_neg = jnp.float32(-1e30)
    masked = jnp.where(lane < NE, logits, big_neg)
    m1 = jnp.max(masked, axis=1, keepdims=True)
    i1 = jnp.min(jnp.where(masked == m1, lane, NE), axis=1, keepdims=True)
    masked2 = jnp.where(lane == i1, big_neg, masked)
    m2 = jnp.max(masked2, axis=1, keepdims=True)
    i2 = jnp.min(jnp.where(masked2 == m2, lane, NE), axis=1, keepdims=True)

    z = jnp.exp(masked - m1)
    s = jnp.sum(z, axis=1, keepdims=True)
    probs = z / s
    gp_ref[...] = probs

    p1 = 1.0 / s
    p2 = jnp.exp(m2 - m1) / s
    denom = p1 + p2
    tki_ref[...] = jnp.concatenate([i1, i2], axis=1)
    tkw_ref[...] = jnp.concatenate([p1 / denom, p2 / denom], axis=1)

    @pl.when(t == 0)
    def _():
        f_ref[...] = jnp.zeros_like(f_ref)
        p_ref[...] = jnp.zeros_like(p_ref)

    f_ref[...] += jnp.sum((lane == i1).astype(jnp.float32), axis=0,
                          keepdims=True)
    p_ref[...] += jnp.sum(probs, axis=0, keepdims=True)

    @pl.when(t == NT - 1)
    def _():
        loss_ref[...] = jnp.reshape(
            (LBW * NE / (T * T)) * jnp.sum(f_ref[...] * p_ref[...]), (1, 1))


def _gate_call(x_flat, gw_pad, gb_pad):
    return pl.pallas_call(
        _gate_body,
        grid=(NT,),
        in_specs=[
            pl.BlockSpec((TG, DM), lambda t: (t, 0)),
            pl.BlockSpec((DM, EP), lambda t: (0, 0)),
            pl.BlockSpec((1, EP), lambda t: (0, 0)),
        ],
        out_specs=[
            pl.BlockSpec((TG, EP), lambda t: (t, 0)),
            pl.BlockSpec((TG, NK), lambda t: (t, 0)),
            pl.BlockSpec((TG, NK), lambda t: (t, 0)),
            pl.BlockSpec((1, EP), lambda t: (0, 0)),
            pl.BlockSpec((1, EP), lambda t: (0, 0)),
            pl.BlockSpec((1, 1), lambda t: (0, 0)),
        ],
        out_shape=[
            jax.ShapeDtypeStruct((T, EP), jnp.float32),
            jax.ShapeDtypeStruct((T, NK), jnp.int32),
            jax.ShapeDtypeStruct((T, NK), jnp.float32),
            jax.ShapeDtypeStruct((1, EP), jnp.float32),
            jax.ShapeDtypeStruct((1, EP), jnp.float32),
            jax.ShapeDtypeStruct((1, 1), jnp.float32),
        ],
        interpret=_INTERPRET,
    )(x_flat, gw_pad, gb_pad)


def _routing_meta(tki, tkw):
    e_p = tki.reshape(-1)
    onehot = (e_p[:, None] == jnp.arange(NE)[None, :]).astype(jnp.int32)
    rank = jnp.sum((jnp.cumsum(onehot, axis=0) - onehot) * onehot, axis=1)
    counts = jnp.sum(onehot, axis=0)
    nblk_e = (counts + BLK - 1) // BLK
    cum = jnp.cumsum(nblk_e).astype(jnp.int32)
    blk_start = jnp.concatenate([jnp.zeros((1,), jnp.int32), cum[:-1]])
    pos = blk_start[e_p] * BLK + rank
    total_blocks = cum[-1]

    bid = jnp.arange(NB, dtype=jnp.int32)
    be_raw = jnp.sum((bid[:, None] >= blk_start[None, :]).astype(jnp.int32),
                     axis=1) - 1
    last_e = jnp.sum((jnp.maximum(total_blocks - 1, 0) >= blk_start)
                     .astype(jnp.int32)) - 1
    valid = bid < total_blocks
    block_expert = jnp.where(valid, be_raw, last_e)
    block_valid = valid.astype(jnp.int32)

    pos2 = pos.reshape(T, NK).astype(jnp.int32)
    inv1 = pos2[:, 0]
    inv2 = pos2[:, 1]
    return block_expert, block_valid, inv1, inv2


def _expert_body(be_ref, bv_ref, x_ref, w1_ref, b1_ref, w2_ref, b2_ref,
                 y_ref):
    b = pl.program_id(0)

    @pl.when(bv_ref[b] == 1)
    def _():
        xb = x_ref[...].astype(jnp.bfloat16)
        h = jnp.dot(xb, w1_ref[0],
                    preferred_element_type=jnp.float32) + b1_ref[0]
        h = 0.5 * h * (1.0 + lax.erf(h * jnp.float32(0.7071067811865476)))
        hb = h.astype(jnp.bfloat16)
        y_ref[...] = jnp.dot(hb, w2_ref[0],
                             preferred_element_type=jnp.float32) + b2_ref[0]


def _expert_call(x_sorted, w1b, b1, w2b, b2, block_expert, block_valid):
    grid_spec = pltpu.PrefetchScalarGridSpec(
        num_scalar_prefetch=2,
        grid=(NB,),
        in_specs=[
            pl.BlockSpec((BLK, DM), lambda b, be, bv: (b, 0)),
            pl.BlockSpec((1, DM, DF), lambda b, be, bv: (be[b], 0, 0)),
            pl.BlockSpec((1, 1, DF), lambda b, be, bv: (be[b], 0, 0)),
            pl.BlockSpec((1, DF, DM), lambda b, be, bv: (be[b], 0, 0)),
            pl.BlockSpec((1, 1, DM), lambda b, be, bv: (be[b], 0, 0)),
        ],
        out_specs=pl.BlockSpec((BLK, DM), lambda b, be, bv: (b, 0)),
    )
    return pl.pallas_call(
        _expert_body,
        grid_spec=grid_spec,
        out_shape=jax.ShapeDtypeStruct((NPAD, DM), jnp.float32),
        compiler_params=pltpu.CompilerParams(
            dimension_semantics=("arbitrary",),
        ),
        interpret=_INTERPRET,
    )(block_expert, block_valid, x_sorted,
      w1b, b1.reshape(NE, 1, DF), w2b, b2.reshape(NE, 1, DM))


TW = T // NW
CT = 32
TCH = TW // CT

@functools.lru_cache(maxsize=None)
def _sc_mesh():
    return plsc.VectorSubcoreMesh(
        core_axis_name="c", subcore_axis_name="s",
        num_cores=NC, num_subcores=NS)


def _sc_dispatch_body(x_hbm, ia_hbm, ib_hbm, out_hbm,
                      ia_v, ib_v, rows_v, sem_a, sem_b):
    wid = lax.axis_index("s") * NC + lax.axis_index("c")

    def chunk(i, carry):
        pltpu.sync_copy(ia_hbm.at[wid, i], ia_v)
        pltpu.sync_copy(ib_hbm.at[wid, i], ib_v)
        pltpu.sync_copy(x_hbm.at[pl.ds(wid * TW + i * CT, CT)], rows_v)
        cp_a = pltpu.async_copy(rows_v, out_hbm.at[ia_v], sem_a)
        cp_b = pltpu.async_copy(rows_v, out_hbm.at[ib_v], sem_b)
        cp_a.wait()
        cp_b.wait()
        return carry

    lax.fori_loop(0, TCH, chunk, 0)


def _sc_dispatch_call(x_flat, inv1, inv2):
    ia = inv1.reshape(NW, TCH, CT)
    ib = inv2.reshape(NW, TCH, CT)
    return pl.kernel(
        _sc_dispatch_body,
        out_type=jax.ShapeDtypeStruct((NPAD, DM), jnp.float32),
        mesh=_sc_mesh(),
        scratch_types=[
            pltpu.VMEM((CT,), jnp.int32),
            pltpu.VMEM((CT,), jnp.int32),
            pltpu.VMEM((CT, DM), jnp.float32),
            pltpu.SemaphoreType.DMA,
            pltpu.SemaphoreType.DMA,
        ],
    )(x_flat, ia, ib)


RC = T // NW
CC = 32


def _sc_combine_body(y_hbm, inv1_hbm, inv2_hbm, wa_hbm, wb_hbm, out_hbm,
                     i1_v, i2_v, wa_v, wb_v, r1_v, r2_v, sem1, sem2):
    wid = lax.axis_index("s") * NC + lax.axis_index("c")
    base = wid * RC

    def chunk(i, carry):
        off = base + i * CC
        pltpu.sync_copy(inv1_hbm.at[pl.ds(off, CC)], i1_v)
        pltpu.sync_copy(inv2_hbm.at[pl.ds(off, CC)], i2_v)
        pltpu.sync_copy(wa_hbm.at[pl.ds(off, CC)], wa_v)
        pltpu.sync_copy(wb_hbm.at[pl.ds(off, CC)], wb_v)
        cp1 = pltpu.async_copy(y_hbm.at[i1_v], r1_v, sem1)
        cp2 = pltpu.async_copy(y_hbm.at[i2_v], r2_v, sem2)
        cp1.wait()
        cp2.wait()

        def row(r, c2):
            wa = wa_v[r, :]
            wb = wb_v[r, :]
            for j in range(DM // 16):
                sl = pl.ds(j * 16, 16)
                r1_v[r, sl] = r1_v[r, sl] * wa + r2_v[r, sl] * wb
            return c2

        lax.fori_loop(0, CC, row, 0)
        pltpu.sync_copy(r1_v, out_hbm.at[pl.ds(off, CC)])
        return carry

    lax.fori_loop(0, RC // CC, chunk, 0)


def _sc_combine_call(y_sorted, inv1, inv2, wa, wb):
    return pl.kernel(
        _sc_combine_body,
        out_type=jax.ShapeDtypeStruct((T, DM), jnp.float32),
        mesh=_sc_mesh(),
        scratch_types=[
            pltpu.VMEM((CC,), jnp.int32),
            pltpu.VMEM((CC,), jnp.int32),
            pltpu.VMEM((CC, 16), jnp.float32),
            pltpu.VMEM((CC, 16), jnp.float32),
            pltpu.VMEM((CC, DM), jnp.float32),
            pltpu.VMEM((CC, DM), jnp.float32),
            pltpu.SemaphoreType.DMA,
            pltpu.SemaphoreType.DMA,
        ],
    )(y_sorted, inv1, inv2, wa, wb)


def kernel(x, gate_W, gate_b, W1, b1, W2, b2):
    B, S, D = x.shape
    x_flat = x.reshape(-1, D)

    gw_pad = jnp.zeros((DM, EP), jnp.float32).at[:, :NE].set(gate_W)
    gb_pad = jnp.zeros((1, EP), jnp.float32).at[0, :NE].set(gate_b)

    gp_pad, tki, tkw, _, _, loss = _gate_call(x_flat, gw_pad, gb_pad)

    block_expert, block_valid, inv1, inv2 = _routing_meta(tki, tkw)

    x_sorted = _sc_dispatch_call(x_flat, inv1, inv2)
    w1b, w2b, _ = lax.optimization_barrier(
        (W1.astype(jnp.bfloat16), W2.astype(jnp.bfloat16), inv1[0]))
    y_sorted = _expert_call(x_sorted, w1b, b1, w2b, b2,
                            block_expert, block_valid)
    out_flat = _sc_combine_call(
        y_sorted, inv1, inv2,
        jnp.broadcast_to(tkw[:, 0:1], (T, 16)),
        jnp.broadcast_to(tkw[:, 1:2], (T, 16)))

    return (out_flat.reshape(B, S, D), loss.reshape(()),
            gp_pad[:, :NE].reshape(B, S, NE), tki.reshape(B, S, NK))


# speedup vs baseline: 1.7834x; 1.0010x over previous
import functools

import jax
import jax.numpy as jnp
from jax import lax
from jax.experimental import pallas as pl
from jax.experimental.pallas import tpu as pltpu
from jax.experimental.pallas import tpu_sc as plsc

DM = 1024
DF = 4096
NE = 8
NK = 2
LBW = 0.01
T = 4096
NP = T * NK

BLK = 512
NB = NP // BLK + NE
NPAD = NB * BLK

EP = 128

NC, NS = 2, 16
NW = NC * NS

_INTERPRET = False


TG = 512
NT = T // TG


def _gate_body(x_ref, gw_ref, gb_ref, gp_ref, tki_ref, tkw_ref,
               f_ref, p_ref, loss_ref):
    t = pl.program_id(0)
    logits = jnp.dot(x_ref[...], gw_ref[...],
                     preferred_element_type=jnp.float32) + gb_ref[...]
    lane = lax.broadcasted_iota(jnp.int32, (TG, EP), 1)
    big_neg = jnp.float32(-1e30)
    masked = jnp.where(lane < NE, logits, big_neg)
    m1 = jnp.max(masked, axis=1, keepdims=True)
    i1 = jnp.min(jnp.where(masked == m1, lane, NE), axis=1, keepdims=True)
    masked2 = jnp.where(lane == i1, big_neg, masked)
    m2 = jnp.max(masked2, axis=1, keepdims=True)
    i2 = jnp.min(jnp.where(masked2 == m2, lane, NE), axis=1, keepdims=True)

    z = jnp.exp(masked - m1)
    s = jnp.sum(z, axis=1, keepdims=True)
    probs = z / s
    gp_ref[...] = probs

    p1 = 1.0 / s
    p2 = jnp.exp(m2 - m1) / s
    denom = p1 + p2
    tki_ref[...] = jnp.concatenate([i1, i2], axis=1)
    tkw_ref[...] = jnp.concatenate([p1 / denom, p2 / denom], axis=1)

    @pl.when(t == 0)
    def _():
        f_ref[...] = jnp.zeros_like(f_ref)
        p_ref[...] = jnp.zeros_like(p_ref)

    f_ref[...] += jnp.sum((lane == i1).astype(jnp.float32), axis=0,
                          keepdims=True)
    p_ref[...] += jnp.sum(probs, axis=0, keepdims=True)

    @pl.when(t == NT - 1)
    def _():
        loss_ref[...] = jnp.reshape(
            (LBW * NE / (T * T)) * jnp.sum(f_ref[...] * p_ref[...]), (1, 1))


def _gate_call(x_flat, gw_pad, gb_pad):
    return pl.pallas_call(
        _gate_body,
        grid=(NT,),
        in_specs=[
            pl.BlockSpec((TG, DM), lambda t: (t, 0)),
            pl.BlockSpec((DM, EP), lambda t: (0, 0)),
            pl.BlockSpec((1, EP), lambda t: (0, 0)),
        ],
        out_specs=[
            pl.BlockSpec((TG, EP), lambda t: (t, 0)),
            pl.BlockSpec((TG, NK), lambda t: (t, 0)),
            pl.BlockSpec((TG, NK), lambda t: (t, 0)),
            pl.BlockSpec((1, EP), lambda t: (0, 0)),
            pl.BlockSpec((1, EP), lambda t: (0, 0)),
            pl.BlockSpec((1, 1), lambda t: (0, 0)),
        ],
        out_shape=[
            jax.ShapeDtypeStruct((T, EP), jnp.float32),
            jax.ShapeDtypeStruct((T, NK), jnp.int32),
            jax.ShapeDtypeStruct((T, NK), jnp.float32),
            jax.ShapeDtypeStruct((1, EP), jnp.float32),
            jax.ShapeDtypeStruct((1, EP), jnp.float32),
            jax.ShapeDtypeStruct((1, 1), jnp.float32),
        ],
        interpret=_INTERPRET,
    )(x_flat, gw_pad, gb_pad)


def _routing_meta(tki, tkw):
    e_p = tki.reshape(-1)
    onehot = (e_p[:, None] == jnp.arange(NE)[None, :]).astype(jnp.int32)
    rank = jnp.sum((jnp.cumsum(onehot, axis=0) - onehot) * onehot, axis=1)
    counts = jnp.sum(onehot, axis=0)
    nblk_e = (counts + BLK - 1) // BLK
    cum = jnp.cumsum(nblk_e).astype(jnp.int32)
    blk_start = jnp.concatenate([jnp.zeros((1,), jnp.int32), cum[:-1]])
    pos = blk_start[e_p] * BLK + rank
    total_blocks = cum[-1]

    bid = jnp.arange(NB, dtype=jnp.int32)
    be_raw = jnp.sum((bid[:, None] >= blk_start[None, :]).astype(jnp.int32),
                     axis=1) - 1
    last_e = jnp.sum((jnp.maximum(total_blocks - 1, 0) >= blk_start)
                     .astype(jnp.int32)) - 1
    valid = bid < total_blocks
    block_expert = jnp.where(valid, be_raw, last_e)
    block_valid = valid.astype(jnp.int32)

    pos2 = pos.reshape(T, NK).astype(jnp.int32)
    inv1 = pos2[:, 0]
    inv2 = pos2[:, 1]
    return block_expert, block_valid, inv1, inv2


def _expert_body(be_ref, bv_ref, x_ref, w1_ref, b1_ref, w2_ref, b2_ref,
                 y_ref):
    b = pl.program_id(0)

    @pl.when(bv_ref[b] == 1)
    def _():
        xb = x_ref[...].astype(jnp.bfloat16)
        h = jnp.dot(xb, w1_ref[0],
                    preferred_element_type=jnp.float32) + b1_ref[0]
        h = 0.5 * h * (1.0 + lax.erf(h * jnp.float32(0.7071067811865476)))
        hb = h.astype(jnp.bfloat16)
        y_ref[...] = jnp.dot(hb, w2_ref[0],
                             preferred_element_type=jnp.float32) + b2_ref[0]


def _expert_call(x_sorted, w1b, b1, w2b, b2, block_expert, block_valid):
    grid_spec = pltpu.PrefetchScalarGridSpec(
        num_scalar_prefetch=2,
        grid=(NB,),
        in_specs=[
            pl.BlockSpec((BLK, DM), lambda b, be, bv: (b, 0)),
            pl.BlockSpec((1, DM, DF), lambda b, be, bv: (be[b], 0, 0)),
            pl.BlockSpec((1, 1, DF), lambda b, be, bv: (be[b], 0, 0)),
            pl.BlockSpec((1, DF, DM), lambda b, be, bv: (be[b], 0, 0)),
            pl.BlockSpec((1, 1, DM), lambda b, be, bv: (be[b], 0, 0)),
        ],
        out_specs=pl.BlockSpec((BLK, DM), lambda b, be, bv: (b, 0)),
    )
    return pl.pallas_call(
        _expert_body,
        grid_spec=grid_spec,
        out_shape=jax.ShapeDtypeStruct((NPAD, DM), jnp.float32),
        compiler_params=pltpu.CompilerParams(
            dimension_semantics=("arbitrary",),
        ),
        interpret=_INTERPRET,
    )(block_expert, block_valid, x_sorted,
      w1b, b1.reshape(NE, 1, DF), w2b, b2.reshape(NE, 1, DM))


TW = T // NW
CT = 64
TCH = TW // CT

@functools.lru_cache(maxsize=None)
def _sc_mesh():
    return plsc.VectorSubcoreMesh(
        core_axis_name="c", subcore_axis_name="s",
        num_cores=NC, num_subcores=NS)


def _sc_dispatch_body(x_hbm, ia_hbm, ib_hbm, out_hbm,
                      ia_v, ib_v, rows_v, sem_a, sem_b):
    wid = lax.axis_index("s") * NC + lax.axis_index("c")

    def chunk(i, carry):
        pltpu.sync_copy(ia_hbm.at[wid, i], ia_v)
        pltpu.sync_copy(ib_hbm.at[wid, i], ib_v)
        pltpu.sync_copy(x_hbm.at[pl.ds(wid * TW + i * CT, CT)], rows_v)
        cp_a = pltpu.async_copy(rows_v, out_hbm.at[ia_v], sem_a)
        cp_b = pltpu.async_copy(rows_v, out_hbm.at[ib_v], sem_b)
        cp_a.wait()
        cp_b.wait()
        return carry

    lax.fori_loop(0, TCH, chunk, 0)


def _sc_dispatch_call(x_flat, inv1, inv2):
    ia = inv1.reshape(NW, TCH, CT)
    ib = inv2.reshape(NW, TCH, CT)
    return pl.kernel(
        _sc_dispatch_body,
        out_type=jax.ShapeDtypeStruct((NPAD, DM), jnp.float32),
        mesh=_sc_mesh(),
        scratch_types=[
            pltpu.VMEM((CT,), jnp.int32),
            pltpu.VMEM((CT,), jnp.int32),
            pltpu.VMEM((CT, DM), jnp.float32),
            pltpu.SemaphoreType.DMA,
            pltpu.SemaphoreType.DMA,
        ],
    )(x_flat, ia, ib)


RC = T // NW
CC = 32


def _sc_combine_body(y_hbm, inv1_hbm, inv2_hbm, wa_hbm, wb_hbm, out_hbm,
                     i1_v, i2_v, wa_v, wb_v, r1_v, r2_v, sem1, sem2):
    wid = lax.axis_index("s") * NC + lax.axis_index("c")
    base = wid * RC

    def chunk(i, carry):
        off = base + i * CC
        pltpu.sync_copy(inv1_hbm.at[pl.ds(off, CC)], i1_v)
        pltpu.sync_copy(inv2_hbm.at[pl.ds(off, CC)], i2_v)
        pltpu.sync_copy(wa_hbm.at[pl.ds(off, CC)], wa_v)
        pltpu.sync_copy(wb_hbm.at[pl.ds(off, CC)], wb_v)
        cp1 = pltpu.async_copy(y_hbm.at[i1_v], r1_v, sem1)
        cp2 = pltpu.async_copy(y_hbm.at[i2_v], r2_v, sem2)
        cp1.wait()
        cp2.wait()

        def row(r, c2):
            wa = wa_v[r, :]
            wb = wb_v[r, :]
            for j in range(DM // 16):
                sl = pl.ds(j * 16, 16)
                r1_v[r, sl] = r1_v[r, sl] * wa + r2_v[r, sl] * wb
            return c2

        lax.fori_loop(0, CC, row, 0)
        pltpu.sync_copy(r1_v, out_hbm.at[pl.ds(off, CC)])
        return carry

    lax.fori_loop(0, RC // CC, chunk, 0)


def _sc_combine_call(y_sorted, inv1, inv2, wa, wb):
    return pl.kernel(
        _sc_combine_body,
        out_type=jax.ShapeDtypeStruct((T, DM), jnp.float32),
        mesh=_sc_mesh(),
        scratch_types=[
            pltpu.VMEM((CC,), jnp.int32),
            pltpu.VMEM((CC,), jnp.int32),
            pltpu.VMEM((CC, 16), jnp.float32),
            pltpu.VMEM((CC, 16), jnp.float32),
            pltpu.VMEM((CC, DM), jnp.float32),
            pltpu.VMEM((CC, DM), jnp.float32),
            pltpu.SemaphoreType.DMA,
            pltpu.SemaphoreType.DMA,
        ],
    )(y_sorted, inv1, inv2, wa, wb)


def kernel(x, gate_W, gate_b, W1, b1, W2, b2):
    B, S, D = x.shape
    x_flat = x.reshape(-1, D)

    gw_pad = jnp.zeros((DM, EP), jnp.float32).at[:, :NE].set(gate_W)
    gb_pad = jnp.zeros((1, EP), jnp.float32).at[0, :NE].set(gate_b)

    gp_pad, tki, tkw, _, _, loss = _gate_call(x_flat, gw_pad, gb_pad)

    block_expert, block_valid, inv1, inv2 = _routing_meta(tki, tkw)

    x_sorted = _sc_dispatch_call(x_flat, inv1, inv2)
    w1b, w2b, _ = lax.optimization_barrier(
        (W1.astype(jnp.bfloat16), W2.astype(jnp.bfloat16), inv1[0]))
    y_sorted = _expert_call(x_sorted, w1b, b1, w2b, b2,
                            block_expert, block_valid)
    out_flat = _sc_combine_call(
        y_sorted, inv1, inv2,
        jnp.broadcast_to(tkw[:, 0:1], (T, 16)),
        jnp.broadcast_to(tkw[:, 1:2], (T, 16)))

    return (out_flat.reshape(B, S, D), loss.reshape(()),
            gp_pad[:, :NE].reshape(B, S, NE), tki.reshape(B, S, NK))
